# baseline (speedup 1.0000x reference)
import functools

import jax
import jax.numpy as jnp
from jax.experimental import pallas as pl
from jax.experimental.pallas import tpu as pltpu

_CLUSTERS = 256
_ITERATIONS = 10
_BITS = 32
_TL = 256


def _body(len_ref, q_ref, k_ref, w_ref, b_ref, out_ref, cnt_ref, grp_ref):
    n = pl.program_id(0)
    L = q_ref.shape[2]
    E = q_ref.shape[3]
    C = _CLUSTERS
    B = _BITS
    T = L // _TL

    q = q_ref[0, 0, :, :]
    k = k_ref[0, 0, :, :]

    proj = jnp.dot(q, w_ref[...], preferred_element_type=jnp.float32)
    proj = proj + b_ref[...]
    bits = (proj > 0).astype(jnp.float32)
    ones_col = jnp.ones((L, 1), dtype=jnp.float32)
    bits_bf = bits.astype(jnp.bfloat16)
    bits_ext = jnp.concatenate([bits, ones_col], axis=1).astype(jnp.bfloat16)
    a_mat = jnp.concatenate([512.0 - 1024.0 * bits, ones_col],
                            axis=1).astype(jnp.bfloat16)
    c_col = jax.lax.broadcasted_iota(jnp.int32, (C, 1), 0).astype(jnp.float32)

    length = jnp.maximum(len_ref[n], 1)
    ntiles = (length + _TL - 1) // _TL
    validf = (jax.lax.broadcasted_iota(jnp.int32, (L, 1), 0) < length)
    validf = validf.astype(jnp.float32)

    row_c = jax.lax.broadcasted_iota(jnp.int32, (C, L), 0) * (L // C)
    col_l = jax.lax.broadcasted_iota(jnp.int32, (C, L), 1)
    sel = (row_c == col_l).astype(jnp.bfloat16)
    cb = jax.lax.dot_general(sel, bits_bf, (((1,), (0,)), ((), ())),
                             preferred_element_type=jnp.float32)

    def onehot_tile(cb_ext, t):
        s = t * _TL
        a_t = a_mat[s:s + _TL, :]
        key = jax.lax.dot_general(a_t, cb_ext, (((1,), (1,)), ((), ())),
                                  preferred_element_type=jnp.float32)
        m = jnp.min(key, axis=1, keepdims=True) - (1.0 - validf[s:s + _TL, :])
        return (key == m).astype(jnp.bfloat16)

    for _ in range(_ITERATIONS):
        cb_ext = jnp.concatenate([cb, c_col], axis=1).astype(jnp.bfloat16)
        cnt_ref[...] = jnp.zeros((C, B + 1), jnp.float32)
        for t in range(T):
            @pl.when(t < ntiles)
            def _(t=t):
                onehot = onehot_tile(cb_ext, t)
                s = t * _TL
                cnt_ref[...] += jax.lax.dot_general(
                    onehot, bits_ext[s:s + _TL, :], (((0,), (0,)), ((), ())),
                    preferred_element_type=jnp.float32)
        cnt = cnt_ref[...]
        member = cnt[:, B:B + 1]
        newcb = (2.0 * cnt[:, :B] > member).astype(jnp.float32)
        cb = jnp.where(member > 0, newcb, cb)

    cb_ext = jnp.concatenate([cb, c_col], axis=1).astype(jnp.bfloat16)
    q_ext = jnp.concatenate([q, ones_col], axis=1).astype(jnp.bfloat16)
    grp_ref[...] = jnp.zeros((C, E + 1), jnp.float32)
    for t in range(T):
        @pl.when(t < ntiles)
        def _(t=t):
            onehot = onehot_tile(cb_ext, t)
            s = t * _TL
            grp_ref[...] += jax.lax.dot_general(
                onehot, q_ext[s:s + _TL, :], (((0,), (0,)), ((), ())),
                preferred_element_type=jnp.float32)
    grp = grp_ref[...]
    counts = grp[:, E:E + 1]
    qg = grp[:, :E] / jnp.maximum(counts, 1.0)
    out_ref[0, 0, :, :] = jax.lax.dot_general(
        qg.astype(jnp.bfloat16), k.astype(jnp.bfloat16),
        (((1,), (1,)), ((), ())),
        preferred_element_type=jnp.float32)


def kernel(queries, keys, attn_mask, query_lengths, planes):
    del attn_mask
    N, L, H, E = queries.shape
    C = _CLUSTERS
    B = _BITS
    w_t = planes[:, :E].T
    bias = planes[:, E].reshape(1, B)
    lengths = query_lengths.astype(jnp.int32)
    qt = jnp.transpose(queries, (0, 2, 1, 3))
    kt = jnp.transpose(keys, (0, 2, 1, 3))

    return pl.pallas_call(
        _body,
        grid=(N, H),
        in_specs=[
            pl.BlockSpec(memory_space=pltpu.SMEM),
            pl.BlockSpec((1, 1, L, E), lambda n, h: (n, h, 0, 0)),
            pl.BlockSpec((1, 1, L, E), lambda n, h: (n, h, 0, 0)),
            pl.BlockSpec((E, B), lambda n, h: (0, 0)),
            pl.BlockSpec((1, B), lambda n, h: (0, 0)),
        ],
        out_specs=pl.BlockSpec((1, 1, C, L), lambda n, h: (n, h, 0, 0)),
        out_shape=jax.ShapeDtypeStruct((N, H, C, L), jnp.float32),
        scratch_shapes=[
            pltpu.VMEM((C, B + 1), jnp.float32),
            pltpu.VMEM((C, E + 1), jnp.float32),
        ],
        compiler_params=pltpu.CompilerParams(
            dimension_semantics=("parallel", "parallel")),
    )(lengths, qt, kt, w_t, bias)


# speedup vs baseline: 1.4809x; 1.4809x over previous
import functools

import jax
import jax.numpy as jnp
from jax.experimental import pallas as pl
from jax.experimental.pallas import tpu as pltpu

_CLUSTERS = 256
_ITERATIONS = 10
_BITS = 32
_TL = 512


def _body(len_ref, q_ref, k_ref, w_ref, b_ref, out_ref, cnt_ref, grp_ref):
    n = pl.program_id(0)
    L = q_ref.shape[2]
    E = q_ref.shape[3]
    C = _CLUSTERS
    B = _BITS
    T = L // _TL

    q = q_ref[0, 0, :, :]
    k = k_ref[0, 0, :, :]

    proj = jnp.dot(q, w_ref[...], preferred_element_type=jnp.float32)
    proj = proj + b_ref[...]
    bits = (proj > 0).astype(jnp.float32)
    ones_col = jnp.ones((L, 1), dtype=jnp.float32)
    bits_bf = bits.astype(jnp.bfloat16)
    bits_ext = jnp.concatenate([bits, ones_col], axis=1).astype(jnp.bfloat16)
    a_mat = jnp.concatenate([512.0 - 1024.0 * bits, ones_col],
                            axis=1).astype(jnp.bfloat16)
    c_col = jax.lax.broadcasted_iota(jnp.int32, (C, 1), 0).astype(jnp.float32)

    length = jnp.maximum(len_ref[n], 1)
    ntiles = (length + _TL - 1) // _TL
    validf = (jax.lax.broadcasted_iota(jnp.int32, (L, 1), 0) < length)
    validf = validf.astype(jnp.float32)

    row_c = jax.lax.broadcasted_iota(jnp.int32, (C, L), 0) * (L // C)
    col_l = jax.lax.broadcasted_iota(jnp.int32, (C, L), 1)
    sel = (row_c == col_l).astype(jnp.bfloat16)
    cb = jax.lax.dot_general(sel, bits_bf, (((1,), (0,)), ((), ())),
                             preferred_element_type=jnp.float32)

    def onehot_tile(cb_ext, t):
        s = t * _TL
        a_t = a_mat[s:s + _TL, :]
        key = jax.lax.dot_general(a_t, cb_ext, (((1,), (1,)), ((), ())),
                                  preferred_element_type=jnp.float32)
        m = jnp.min(key, axis=1, keepdims=True) - (1.0 - validf[s:s + _TL, :])
        return (key == m).astype(jnp.bfloat16)

    for _ in range(_ITERATIONS):
        cb_ext = jnp.concatenate([cb, c_col], axis=1).astype(jnp.bfloat16)
        cnt_ref[...] = jnp.zeros((C, B + 1), jnp.float32)
        for t in range(T):
            @pl.when(t < ntiles)
            def _(t=t):
                onehot = onehot_tile(cb_ext, t)
                s = t * _TL
                cnt_ref[...] += jax.lax.dot_general(
                    onehot, bits_ext[s:s + _TL, :], (((0,), (0,)), ((), ())),
                    preferred_element_type=jnp.float32)
        cnt = cnt_ref[...]
        member = cnt[:, B:B + 1]
        newcb = (2.0 * cnt[:, :B] > member).astype(jnp.float32)
        cb = jnp.where(member > 0, newcb, cb)

    cb_ext = jnp.concatenate([cb, c_col], axis=1).astype(jnp.bfloat16)
    q_ext = jnp.concatenate([q, ones_col], axis=1).astype(jnp.bfloat16)
    grp_ref[...] = jnp.zeros((C, E + 1), jnp.float32)
    for t in range(T):
        @pl.when(t < ntiles)
        def _(t=t):
            onehot = onehot_tile(cb_ext, t)
            s = t * _TL
            grp_ref[...] += jax.lax.dot_general(
                onehot, q_ext[s:s + _TL, :], (((0,), (0,)), ((), ())),
                preferred_element_type=jnp.float32)
    grp = grp_ref[...]
    counts = grp[:, E:E + 1]
    qg = grp[:, :E] / jnp.maximum(counts, 1.0)
    out_ref[0, 0, :, :] = jax.lax.dot_general(
        qg.astype(jnp.bfloat16), k.astype(jnp.bfloat16),
        (((1,), (1,)), ((), ())),
        preferred_element_type=jnp.float32)


def kernel(queries, keys, attn_mask, query_lengths, planes):
    del attn_mask
    N, L, H, E = queries.shape
    C = _CLUSTERS
    B = _BITS
    w_t = planes[:, :E].T
    bias = planes[:, E].reshape(1, B)
    lengths = query_lengths.astype(jnp.int32)
    qt = jnp.transpose(queries, (0, 2, 1, 3))
    kt = jnp.transpose(keys, (0, 2, 1, 3))

    return pl.pallas_call(
        _body,
        grid=(N, H),
        in_specs=[
            pl.BlockSpec(memory_space=pltpu.SMEM),
            pl.BlockSpec((1, 1, L, E), lambda n, h: (n, h, 0, 0)),
            pl.BlockSpec((1, 1, L, E), lambda n, h: (n, h, 0, 0)),
            pl.BlockSpec((E, B), lambda n, h: (0, 0)),
            pl.BlockSpec((1, B), lambda n, h: (0, 0)),
        ],
        out_specs=pl.BlockSpec((1, 1, C, L), lambda n, h: (n, h, 0, 0)),
        out_shape=jax.ShapeDtypeStruct((N, H, C, L), jnp.float32),
        scratch_shapes=[
            pltpu.VMEM((C, B + 1), jnp.float32),
            pltpu.VMEM((C, E + 1), jnp.float32),
        ],
        compiler_params=pltpu.CompilerParams(
            dimension_semantics=("parallel", "parallel")),
    )(lengths, qt, kt, w_t, bias)


# speedup vs baseline: 1.6741x; 1.1305x over previous
import functools

import jax
import jax.numpy as jnp
from jax.experimental import pallas as pl
from jax.experimental.pallas import tpu as pltpu

_CLUSTERS = 256
_ITERATIONS = 10
_BITS = 32
_BUCKET = 512


def _body(l_eff, len_ref, q_ref, k_ref, w_ref, b_ref, out_ref):
    L = q_ref.shape[1]
    E = q_ref.shape[2]
    C = _CLUSTERS
    B = _BITS

    q = q_ref[0, :, :]
    k = k_ref[0, :, :]

    proj = jnp.dot(q, w_ref[...], preferred_element_type=jnp.float32)
    proj = proj + b_ref[...]
    bits = (proj > 0).astype(jnp.float32)
    ones_col = jnp.ones((L, 1), dtype=jnp.float32)
    bits_bf = bits.astype(jnp.bfloat16)
    bits_ext = jnp.concatenate([bits[:l_eff], ones_col[:l_eff]],
                               axis=1).astype(jnp.bfloat16)
    a_mat = jnp.concatenate([512.0 - 1024.0 * bits[:l_eff], ones_col[:l_eff]],
                            axis=1).astype(jnp.bfloat16)
    c_col = jax.lax.broadcasted_iota(jnp.int32, (C, 1), 0).astype(jnp.float32)

    length = jnp.maximum(len_ref[0], 1)
    validf = (jax.lax.broadcasted_iota(jnp.int32, (l_eff, 1), 0) < length)
    validf = validf.astype(jnp.float32)

    row_c = jax.lax.broadcasted_iota(jnp.int32, (C, L), 0) * (L // C)
    col_l = jax.lax.broadcasted_iota(jnp.int32, (C, L), 1)
    sel = (row_c == col_l).astype(jnp.bfloat16)
    cb = jax.lax.dot_general(sel, bits_bf, (((1,), (0,)), ((), ())),
                             preferred_element_type=jnp.float32)

    def onehot_bf(cb):
        cb_ext = jnp.concatenate([cb, c_col], axis=1).astype(jnp.bfloat16)
        key = jax.lax.dot_general(a_mat, cb_ext, (((1,), (1,)), ((), ())),
                                  preferred_element_type=jnp.float32)
        m = jnp.min(key, axis=1, keepdims=True) - (1.0 - validf)
        return (key == m).astype(jnp.bfloat16)

    for _ in range(_ITERATIONS):
        onehot = onehot_bf(cb)
        cnt = jax.lax.dot_general(onehot, bits_ext, (((0,), (0,)), ((), ())),
                                  preferred_element_type=jnp.float32)
        member = cnt[:, B:B + 1]
        newcb = (2.0 * cnt[:, :B] > member).astype(jnp.float32)
        cb = jnp.where(member > 0, newcb, cb)

    onehot = onehot_bf(cb)

    q_ext = jnp.concatenate([q[:l_eff], ones_col[:l_eff]],
                            axis=1).astype(jnp.bfloat16)
    grp = jax.lax.dot_general(onehot, q_ext, (((0,), (0,)), ((), ())),
                              preferred_element_type=jnp.float32)
    counts = grp[:, E:E + 1]
    qg = grp[:, :E] / jnp.maximum(counts, 1.0)
    out_ref[0, :, :] = jax.lax.dot_general(
        qg.astype(jnp.bfloat16), k.astype(jnp.bfloat16),
        (((1,), (1,)), ((), ())),
        preferred_element_type=jnp.float32)


def _make_call(l_eff, H, L, E):
    C = _CLUSTERS
    B = _BITS
    return pl.pallas_call(
        functools.partial(_body, l_eff),
        grid=(H,),
        in_specs=[
            pl.BlockSpec(memory_space=pltpu.SMEM),
            pl.BlockSpec((1, L, E), lambda h: (h, 0, 0)),
            pl.BlockSpec((1, L, E), lambda h: (h, 0, 0)),
            pl.BlockSpec((E, B), lambda h: (0, 0)),
            pl.BlockSpec((1, B), lambda h: (0, 0)),
        ],
        out_specs=pl.BlockSpec((1, C, L), lambda h: (h, 0, 0)),
        out_shape=jax.ShapeDtypeStruct((H, C, L), jnp.float32),
        compiler_params=pltpu.CompilerParams(
            dimension_semantics=("parallel",)),
    )


def kernel(queries, keys, attn_mask, query_lengths, planes):
    del attn_mask
    N, L, H, E = queries.shape
    B = _BITS
    w_t = planes[:, :E].T
    bias = planes[:, E].reshape(1, B)
    lengths = jnp.maximum(query_lengths.astype(jnp.int32), 1)
    qt = jnp.transpose(queries, (0, 2, 1, 3))
    kt = jnp.transpose(keys, (0, 2, 1, 3))

    n_buckets = L // _BUCKET
    calls = [_make_call(_BUCKET * (i + 1), H, L, E) for i in range(n_buckets)]
    branches = [
        (lambda call: lambda ln, qn, kn: call(ln, qn, kn, w_t, bias))(c)
        for c in calls
    ]

    outs = []
    for n in range(N):
        ln = lengths[n].reshape(1)
        idx = jnp.clip((lengths[n] - 1) // _BUCKET, 0, n_buckets - 1)
        outs.append(jax.lax.switch(idx, branches, ln, qt[n], kt[n]))
    return jnp.stack(outs, axis=0)
